# Initial kernel scaffold; baseline (speedup 1.0000x reference)
import jax
import jax.numpy as jnp
from jax.experimental import pallas as pl


def kernel(local_preds, idx, edge_index):
    raise NotImplementedError("write your pallas kernel here")


# speedup vs baseline: 9.3727x; 9.3727x over previous
import functools

import jax
import jax.numpy as jnp
from jax import lax
from jax.experimental import pallas as pl
from jax.experimental.pallas import tpu as pltpu
from jax.experimental.pallas import tpu_sc as plsc

N = 10000
E = 160000
D = 256
N_IDX = 2000
ALPHA = 0.1
BETA = 1.0
NITER = 10

AB1 = ALPHA * BETA + 1.0
CM = (ALPHA * BETA + 1.0 - ALPHA) / AB1
C1 = (1.0 + BETA) / AB1
C2 = BETA / AB1

NSC = 2
NTILE = 16
Q = 128
NP = 10240
NPH = 2
PH = NP // NPH
RPP = PH // NTILE
ZC = 64
NZ = RPP // ZC
CHUNK = 128
NCB = 80
SHARE_RND = 8 * CHUNK
NTOT_CH = 1600
GROW = N + 16
SROW = PH + 16
ACCR = PH + CHUNK
NIDXP = 2048


def _sc_body(s0h, colsa, lrowsa, d2a, kcha, idxp, sdegp,
             xstate, xalt, sig1, outp,
             colbuf, rowbuf, gbufs, abuf, sbuf, obuf, zbuf,
             d2v, sdegv, kv, idxbuf, acc, sems, ssems, osem):
    cid = lax.axis_index("c")
    sid = lax.axis_index("s")

    for r in range(NPH):
        pltpu.sync_copy(d2a.at[pl.ds(r * PH + sid * RPP, RPP)],
                        d2v.at[pl.ds(r * RPP, RPP)])
    pltpu.sync_copy(kcha, kv.at[pl.ds(0, 4 * NTILE)])

    def _zrow(i, _):
        for q in range(Q // 16):
            zbuf[i, pl.ds(q * 16, 16)] = jnp.zeros((16,), jnp.float32)
        return 0
    lax.fori_loop(0, ZC, _zrow, 0)
    for z in range(ACCR // NTILE // ZC + 1):
        base = sid * (ACCR // NTILE)
        pltpu.sync_copy(zbuf, acc.at[pl.ds(jnp.minimum(base + z * ZC,
                                                       ACCR - ZC), ZC)])
    plsc.subcore_barrier()

    def amul_phase(src_q, r):
        cb = pl.multiple_of(kv[pl.ds(2 * NTILE + r * NTILE + sid, 16)][0], 8)
        pltpu.sync_copy(colsa.at[pl.ds(cb, NCB)], colbuf)
        pltpu.sync_copy(lrowsa.at[pl.ds(cb, NCB)], rowbuf)
        kch = kv[pl.ds(r * NTILE + sid, 16)][0]

        @pl.when(kch > 0)
        def _():
            pltpu.async_copy(src_q.at[colbuf.at[0]], gbufs.at[0], sems.at[0])

            def chunk_body(j, _):
                slot = lax.rem(j, 2)
                pltpu.make_async_copy(src_q.at[colbuf.at[j]],
                                      gbufs.at[slot], sems.at[slot]).wait()

                @pl.when(j >= 1)
                def _():
                    pltpu.make_async_copy(
                        gbufs.at[1 - slot], acc.at[rowbuf.at[j - 1]],
                        ssems.at[1 - slot]).wait()

                @pl.when(j + 1 < kch)
                def _():
                    pltpu.async_copy(src_q.at[colbuf.at[j + 1]],
                                     gbufs.at[1 - slot], sems.at[1 - slot])
                pltpu.async_copy(gbufs.at[slot], acc.at[rowbuf.at[j]],
                                 ssems.at[slot], add=True)
                return 0
            lax.fori_loop(0, kch, chunk_body, 0)
            last = lax.rem(kch - 1, 2)
            pltpu.make_async_copy(gbufs.at[last], acc.at[rowbuf.at[kch - 1]],
                                  ssems.at[last]).wait()

    def elementwise_phase(src_q, dst_state, qid, r, stage1):
        for z in range(NZ):
            la = sid * RPP + z * ZC
            rows = pl.ds(r * PH + la, ZC)
            pltpu.sync_copy(acc.at[pl.ds(la, ZC)], abuf)
            pltpu.sync_copy(src_q.at[rows], sbuf)
            if not stage1:
                pltpu.sync_copy(sig1.at[qid, rows], obuf)

            def row_body(i, _):
                d2 = d2v[pl.ds(r * RPP + z * ZC + i, 16)][0]
                for q in range(Q // 16):
                    sl = pl.ds(q * 16, 16)
                    t = abuf[i, sl] + sbuf[i, sl]
                    if stage1:
                        sbuf[i, sl] = C1 * sbuf[i, sl] - (C2 * d2) * t
                    else:
                        sbuf[i, sl] = (CM * d2) * t + ALPHA * obuf[i, sl]
                return 0
            lax.fori_loop(0, ZC, row_body, 0)

            if stage1:
                pltpu.sync_copy(sbuf, sig1.at[qid, rows])
            pltpu.sync_copy(sbuf, dst_state.at[qid, rows])
            pltpu.sync_copy(zbuf, acc.at[pl.ds(la, ZC)])

    def sweep(src, dst_state, stage1):
        qid = cid
        for r in range(NPH):
            amul_phase(src.at[qid], r)
            plsc.subcore_barrier()
            elementwise_phase(src.at[qid], dst_state, qid, r, stage1)
            plsc.subcore_barrier()

    sweep(s0h, xstate, stage1=True)

    def iter_body(i, carry):
        @pl.when(lax.rem(i, 2) == 0)
        def _():
            sweep(xstate, xalt, stage1=False)

        @pl.when(lax.rem(i, 2) == 1)
        def _():
            sweep(xalt, xstate, stage1=False)
        return carry
    lax.fori_loop(0, NITER, iter_body, 0)

    pltpu.sync_copy(idxp.at[pl.ds(sid * CHUNK, CHUNK)], idxbuf)
    pltpu.sync_copy(sdegp.at[pl.ds(sid * CHUNK, CHUNK)],
                    sdegv.at[pl.ds(0, CHUNK)])
    pltpu.async_copy(xstate.at[cid].at[idxbuf], gbufs.at[0], osem).wait()

    def out_row(i, _):
        sd = sdegv[pl.ds(i, 16)][0]
        for q in range(Q // 16):
            sl = pl.ds(q * 16, 16)
            gbufs[0, i, sl] = sd * gbufs[0, i, sl]
        return 0
    lax.fori_loop(0, CHUNK, out_row, 0)
    pltpu.sync_copy(gbufs.at[0], outp.at[cid, pl.ds(sid * CHUNK, CHUNK)])


_sc_call = functools.partial(
    pl.kernel,
    out_type=[
        jax.ShapeDtypeStruct((NSC, NP, Q), jnp.float32),
        jax.ShapeDtypeStruct((NSC, NP, Q), jnp.float32),
        jax.ShapeDtypeStruct((NSC, NP, Q), jnp.float32),
        jax.ShapeDtypeStruct((NSC, NIDXP, Q), jnp.float32),
    ],
    mesh=plsc.VectorSubcoreMesh(core_axis_name="c", subcore_axis_name="s",
                                num_cores=NSC),
    scratch_types=[
        pltpu.VMEM((NCB, CHUNK), jnp.int32),
        pltpu.VMEM((NCB, CHUNK), jnp.int32),
        pltpu.VMEM((2, CHUNK, Q), jnp.float32),
        pltpu.VMEM((ZC, Q), jnp.float32),
        pltpu.VMEM((ZC, Q), jnp.float32),
        pltpu.VMEM((ZC, Q), jnp.float32),
        pltpu.VMEM((ZC, Q), jnp.float32),
        pltpu.VMEM((2 * RPP + 16,), jnp.float32),
        pltpu.VMEM((CHUNK + 16,), jnp.float32),
        pltpu.VMEM((4 * NTILE + 16,), jnp.int32),
        pltpu.VMEM((CHUNK,), jnp.int32),
        pltpu.VMEM_SHARED((ACCR, Q), jnp.float32),
        pltpu.SemaphoreType.DMA((2,)),
        pltpu.SemaphoreType.DMA((2,)),
        pltpu.SemaphoreType.DMA,
    ],
)(_sc_body)


def _ceil_div(a, b):
    return (a + b - 1) // b


def kernel(local_preds, idx, edge_index):
    src = edge_index[0].astype(jnp.int32)
    dst = edge_index[1].astype(jnp.int32)

    deg = jax.ops.segment_sum(jnp.ones((E,), jnp.float32), src,
                              num_segments=N) + 1.0
    d2 = 1.0 / deg
    dinv = lax.rsqrt(deg)
    sdeg = deg * dinv

    s0 = dinv[:, None] * local_preds
    s0h = jnp.pad(s0.reshape(N, NSC, Q).transpose(1, 0, 2),
                  ((0, 0), (0, NP - N), (0, 0)))

    phase = (src >= PH).astype(jnp.int32)
    e1 = jnp.sum(phase)
    e0 = E - e1
    c1r = jnp.cumsum(phase) - phase
    c0r = jnp.arange(E, dtype=jnp.int32) - c1r
    rank = jnp.where(phase == 1, c1r, c0r).astype(jnp.int32)
    n0 = SHARE_RND * _ceil_div(e0, NTILE * SHARE_RND)
    n1 = SHARE_RND * _ceil_div(e1, NTILE * SHARE_RND)
    nshare = jnp.where(phase == 1, n1, n0)
    tile = rank // jnp.maximum(nshare, 1)
    within = rank - tile * nshare
    pos = phase * (NTILE * n0) + tile * nshare + within
    colsP = jnp.full((NTOT_CH * CHUNK,), GROW, jnp.int32)
    colsP = colsP.at[pos].add(dst - GROW, unique_indices=True)
    lrowsP = jnp.full((NTOT_CH * CHUNK,), SROW, jnp.int32)
    lrowsP = lrowsP.at[pos].add(src - phase * PH - SROW, unique_indices=True)
    colsa = colsP.reshape(NTOT_CH, CHUNK)
    lrowsa = lrowsP.reshape(NTOT_CH, CHUNK)
    t16 = jnp.arange(NTILE, dtype=jnp.int32)
    c0t = jnp.clip(e0 - t16 * n0, 0, n0)
    c1t = jnp.clip(e1 - t16 * n1, 0, n1)
    counts = _ceil_div(jnp.concatenate([c0t, c1t]), CHUNK).astype(jnp.int32)
    cb0 = t16 * (n0 // CHUNK)
    cb1 = NTILE * (n0 // CHUNK) + t16 * (n1 // CHUNK)
    kcha = jnp.concatenate([counts, cb0, cb1]).astype(jnp.int32)

    d2a = jnp.pad(d2, (0, NP - N))
    idxp = jnp.pad(idx.astype(jnp.int32), (0, NIDXP - N_IDX))
    sdegp = jnp.pad(sdeg[idx], (0, NIDXP - N_IDX))

    _, _, _, outp = _sc_call(s0h, colsa, lrowsa, d2a, kcha, idxp, sdegp)
    return outp.transpose(1, 0, 2).reshape(NIDXP, D)[:N_IDX]


# speedup vs baseline: 9.8526x; 1.0512x over previous
import functools

import jax
import jax.numpy as jnp
from jax import lax
from jax.experimental import pallas as pl
from jax.experimental.pallas import tpu as pltpu
from jax.experimental.pallas import tpu_sc as plsc

N = 10000
E = 160000
D = 256
N_IDX = 2000
ALPHA = 0.1
BETA = 1.0
NITER = 10

AB1 = ALPHA * BETA + 1.0
CM = (ALPHA * BETA + 1.0 - ALPHA) / AB1
C1 = (1.0 + BETA) / AB1
C2 = BETA / AB1

NSC = 2
NTILE = 16
Q = 128
NP = 10240
NPH = 2
PH = NP // NPH
RPP = PH // NTILE
ZC = 64
NZ = RPP // ZC
CHUNK = 128
NCB = 80
SHARE_RND = 8 * CHUNK
NTOT_CH = 1600
GROW = N + 16
SROW = PH + 16
ACCR = PH + CHUNK
NIDXP = 2048


def _sc_body(s0h, colsa, lrowsa, d2a, kcha, idxp, sdegp,
             xstate, xalt, sig1, outp,
             colbuf, rowbuf, gbufs, abuf, sbuf, obuf, zbuf,
             d2v, sdegv, kv, idxbuf, acc, sems, ssems, esems, osem):
    cid = lax.axis_index("c")
    sid = lax.axis_index("s")

    for r in range(NPH):
        pltpu.sync_copy(d2a.at[pl.ds(r * PH + sid * RPP, RPP)],
                        d2v.at[pl.ds(r * RPP, RPP)])
    pltpu.sync_copy(kcha, kv.at[pl.ds(0, 4 * NTILE)])

    def _zrow(i, _):
        for q in range(Q // 16):
            zbuf[i, pl.ds(q * 16, 16)] = jnp.zeros((16,), jnp.float32)
        return 0
    lax.fori_loop(0, ZC, _zrow, 0)
    for z in range(ACCR // NTILE // ZC + 1):
        base = sid * (ACCR // NTILE)
        pltpu.sync_copy(zbuf, acc.at[pl.ds(jnp.minimum(base + z * ZC,
                                                       ACCR - ZC), ZC)])
    plsc.subcore_barrier()

    def amul_phase(src_q, r):
        cb = pl.multiple_of(kv[pl.ds(2 * NTILE + r * NTILE + sid, 16)][0], 8)
        pltpu.sync_copy(colsa.at[pl.ds(cb, NCB)], colbuf)
        pltpu.sync_copy(lrowsa.at[pl.ds(cb, NCB)], rowbuf)
        kch = kv[pl.ds(r * NTILE + sid, 16)][0]

        @pl.when(kch > 0)
        def _():
            pltpu.async_copy(src_q.at[colbuf.at[0]], gbufs.at[0], sems.at[0])

            def chunk_body(j, _):
                slot = lax.rem(j, 2)
                pltpu.make_async_copy(src_q.at[colbuf.at[j]],
                                      gbufs.at[slot], sems.at[slot]).wait()

                @pl.when(j >= 1)
                def _():
                    pltpu.make_async_copy(
                        gbufs.at[1 - slot], acc.at[rowbuf.at[j - 1]],
                        ssems.at[1 - slot]).wait()

                @pl.when(j + 1 < kch)
                def _():
                    pltpu.async_copy(src_q.at[colbuf.at[j + 1]],
                                     gbufs.at[1 - slot], sems.at[1 - slot])
                pltpu.async_copy(gbufs.at[slot], acc.at[rowbuf.at[j]],
                                 ssems.at[slot], add=True)
                return 0
            lax.fori_loop(0, kch, chunk_body, 0)
            last = lax.rem(kch - 1, 2)
            pltpu.make_async_copy(gbufs.at[last], acc.at[rowbuf.at[kch - 1]],
                                  ssems.at[last]).wait()

    def elementwise_phase(src_q, dst_state, qid, r, stage1):
        for z in range(NZ):
            la = sid * RPP + z * ZC
            rows = pl.ds(r * PH + la, ZC)
            cpa = pltpu.async_copy(acc.at[pl.ds(la, ZC)], abuf, esems.at[0])
            cps = pltpu.async_copy(src_q.at[rows], sbuf, esems.at[1])
            if not stage1:
                pltpu.async_copy(sig1.at[qid, rows], obuf, osem).wait()
            cpa.wait()
            cps.wait()

            def row_body(i, _):
                d2 = d2v[pl.ds(r * RPP + z * ZC + i, 16)][0]
                for q in range(Q // 16):
                    sl = pl.ds(q * 16, 16)
                    t = abuf[i, sl] + sbuf[i, sl]
                    if stage1:
                        sbuf[i, sl] = C1 * sbuf[i, sl] - (C2 * d2) * t
                    else:
                        sbuf[i, sl] = (CM * d2) * t + ALPHA * obuf[i, sl]
                return 0
            lax.fori_loop(0, ZC, row_body, 0)

            if stage1:
                pltpu.sync_copy(sbuf, sig1.at[qid, rows])
            pltpu.sync_copy(sbuf, dst_state.at[qid, rows])
            pltpu.sync_copy(zbuf, acc.at[pl.ds(la, ZC)])

    def sweep(src, dst_state, stage1):
        qid = cid
        for r in range(NPH):
            amul_phase(src.at[qid], r)
            plsc.subcore_barrier()
            elementwise_phase(src.at[qid], dst_state, qid, r, stage1)
            plsc.subcore_barrier()

    sweep(s0h, xstate, stage1=True)

    def iter_body(i, carry):
        @pl.when(lax.rem(i, 2) == 0)
        def _():
            sweep(xstate, xalt, stage1=False)

        @pl.when(lax.rem(i, 2) == 1)
        def _():
            sweep(xalt, xstate, stage1=False)
        return carry
    lax.fori_loop(0, NITER, iter_body, 0)

    pltpu.sync_copy(idxp.at[pl.ds(sid * CHUNK, CHUNK)], idxbuf)
    pltpu.sync_copy(sdegp.at[pl.ds(sid * CHUNK, CHUNK)],
                    sdegv.at[pl.ds(0, CHUNK)])
    pltpu.async_copy(xstate.at[cid].at[idxbuf], gbufs.at[0], osem).wait()

    def out_row(i, _):
        sd = sdegv[pl.ds(i, 16)][0]
        for q in range(Q // 16):
            sl = pl.ds(q * 16, 16)
            gbufs[0, i, sl] = sd * gbufs[0, i, sl]
        return 0
    lax.fori_loop(0, CHUNK, out_row, 0)
    pltpu.sync_copy(gbufs.at[0], outp.at[cid, pl.ds(sid * CHUNK, CHUNK)])


_sc_call = functools.partial(
    pl.kernel,
    out_type=[
        jax.ShapeDtypeStruct((NSC, NP, Q), jnp.float32),
        jax.ShapeDtypeStruct((NSC, NP, Q), jnp.float32),
        jax.ShapeDtypeStruct((NSC, NP, Q), jnp.float32),
        jax.ShapeDtypeStruct((NSC, NIDXP, Q), jnp.float32),
    ],
    mesh=plsc.VectorSubcoreMesh(core_axis_name="c", subcore_axis_name="s",
                                num_cores=NSC),
    scratch_types=[
        pltpu.VMEM((NCB, CHUNK), jnp.int32),
        pltpu.VMEM((NCB, CHUNK), jnp.int32),
        pltpu.VMEM((2, CHUNK, Q), jnp.float32),
        pltpu.VMEM((ZC, Q), jnp.float32),
        pltpu.VMEM((ZC, Q), jnp.float32),
        pltpu.VMEM((ZC, Q), jnp.float32),
        pltpu.VMEM((ZC, Q), jnp.float32),
        pltpu.VMEM((2 * RPP + 16,), jnp.float32),
        pltpu.VMEM((CHUNK + 16,), jnp.float32),
        pltpu.VMEM((4 * NTILE + 16,), jnp.int32),
        pltpu.VMEM((CHUNK,), jnp.int32),
        pltpu.VMEM_SHARED((ACCR, Q), jnp.float32),
        pltpu.SemaphoreType.DMA((2,)),
        pltpu.SemaphoreType.DMA((2,)),
        pltpu.SemaphoreType.DMA((2,)),
        pltpu.SemaphoreType.DMA,
    ],
)(_sc_body)


def _ceil_div(a, b):
    return (a + b - 1) // b


def kernel(local_preds, idx, edge_index):
    src = edge_index[0].astype(jnp.int32)
    dst = edge_index[1].astype(jnp.int32)

    deg = jax.ops.segment_sum(jnp.ones((E,), jnp.float32), src,
                              num_segments=N) + 1.0
    d2 = 1.0 / deg
    dinv = lax.rsqrt(deg)
    sdeg = deg * dinv

    s0 = dinv[:, None] * local_preds
    s0h = jnp.pad(s0.reshape(N, NSC, Q).transpose(1, 0, 2),
                  ((0, 0), (0, NP - N), (0, 0)))

    phase = (src >= PH).astype(jnp.int32)
    e1 = jnp.sum(phase)
    e0 = E - e1
    c1r = jnp.cumsum(phase) - phase
    c0r = jnp.arange(E, dtype=jnp.int32) - c1r
    rank = jnp.where(phase == 1, c1r, c0r).astype(jnp.int32)
    n0 = SHARE_RND * _ceil_div(e0, NTILE * SHARE_RND)
    n1 = SHARE_RND * _ceil_div(e1, NTILE * SHARE_RND)
    nshare = jnp.where(phase == 1, n1, n0)
    tile = rank // jnp.maximum(nshare, 1)
    within = rank - tile * nshare
    pos = phase * (NTILE * n0) + tile * nshare + within
    colsP = jnp.full((NTOT_CH * CHUNK,), GROW, jnp.int32)
    colsP = colsP.at[pos].add(dst - GROW, unique_indices=True)
    lrowsP = jnp.full((NTOT_CH * CHUNK,), SROW, jnp.int32)
    lrowsP = lrowsP.at[pos].add(src - phase * PH - SROW, unique_indices=True)
    colsa = colsP.reshape(NTOT_CH, CHUNK)
    lrowsa = lrowsP.reshape(NTOT_CH, CHUNK)
    t16 = jnp.arange(NTILE, dtype=jnp.int32)
    c0t = jnp.clip(e0 - t16 * n0, 0, n0)
    c1t = jnp.clip(e1 - t16 * n1, 0, n1)
    counts = _ceil_div(jnp.concatenate([c0t, c1t]), CHUNK).astype(jnp.int32)
    cb0 = t16 * (n0 // CHUNK)
    cb1 = NTILE * (n0 // CHUNK) + t16 * (n1 // CHUNK)
    kcha = jnp.concatenate([counts, cb0, cb1]).astype(jnp.int32)

    d2a = jnp.pad(d2, (0, NP - N))
    idxp = jnp.pad(idx.astype(jnp.int32), (0, NIDXP - N_IDX))
    sdegp = jnp.pad(sdeg[idx], (0, NIDXP - N_IDX))

    _, _, _, outp = _sc_call(s0h, colsa, lrowsa, d2a, kcha, idxp, sdegp)
    return outp.transpose(1, 0, 2).reshape(NIDXP, D)[:N_IDX]


# speedup vs baseline: 10.0946x; 1.0246x over previous
import functools

import jax
import jax.numpy as jnp
from jax import lax
from jax.experimental import pallas as pl
from jax.experimental.pallas import tpu as pltpu
from jax.experimental.pallas import tpu_sc as plsc

N = 10000
E = 160000
D = 256
N_IDX = 2000
ALPHA = 0.1
BETA = 1.0
NITER = 10

AB1 = ALPHA * BETA + 1.0
CM = (ALPHA * BETA + 1.0 - ALPHA) / AB1
C1 = (1.0 + BETA) / AB1
C2 = BETA / AB1

NSC = 2
NTILE = 16
Q = 128
NP = 10240
NPH = 2
PH = NP // NPH
RPP = PH // NTILE
ZC = 64
NZ = RPP // ZC
CHUNK = 128
NCB = 80
SHARE_RND = 8 * CHUNK
NTOT_CH = 1600
GROW = N + 16
SROW = PH + 16
ACCR = PH + CHUNK
NIDXP = 2048


def _sc_body(s0h, colsa, lrowsa, d2a, kcha, idxp, sdegp,
             xstate, xalt, sig1, outp,
             colbuf, rowbuf, gbufs, abuf, sbuf, obuf, zbuf,
             d2v, sdegv, kv, idxbuf, acc, sems, ssems, esems, zsem, osem):
    cid = lax.axis_index("c")
    sid = lax.axis_index("s")

    for r in range(NPH):
        pltpu.sync_copy(d2a.at[pl.ds(r * PH + sid * RPP, RPP)],
                        d2v.at[pl.ds(r * RPP, RPP)])
    pltpu.sync_copy(kcha, kv.at[pl.ds(0, 4 * NTILE)])

    def _zrow(i, _):
        for q in range(Q // 16):
            zbuf[i, pl.ds(q * 16, 16)] = jnp.zeros((16,), jnp.float32)
        return 0
    lax.fori_loop(0, ZC, _zrow, 0)
    for z in range(ACCR // NTILE // ZC + 1):
        base = sid * (ACCR // NTILE)
        pltpu.sync_copy(zbuf, acc.at[pl.ds(jnp.minimum(base + z * ZC,
                                                       ACCR - ZC), ZC)])
    plsc.subcore_barrier()

    def amul_phase(src_q, r):
        cb = pl.multiple_of(kv[pl.ds(2 * NTILE + r * NTILE + sid, 16)][0], 8)
        cpc = pltpu.async_copy(colsa.at[pl.ds(cb, NCB)], colbuf, esems.at[0])
        cpr = pltpu.async_copy(lrowsa.at[pl.ds(cb, NCB)], rowbuf, esems.at[1])
        kch = kv[pl.ds(r * NTILE + sid, 16)][0]
        cpc.wait()
        cpr.wait()

        @pl.when(kch > 0)
        def _():
            pltpu.async_copy(src_q.at[colbuf.at[0]], gbufs.at[0], sems.at[0])

            def chunk_body(j, _):
                slot = lax.rem(j, 2)
                pltpu.make_async_copy(src_q.at[colbuf.at[j]],
                                      gbufs.at[slot], sems.at[slot]).wait()

                @pl.when(j >= 1)
                def _():
                    pltpu.make_async_copy(
                        gbufs.at[1 - slot], acc.at[rowbuf.at[j - 1]],
                        ssems.at[1 - slot]).wait()

                @pl.when(j + 1 < kch)
                def _():
                    pltpu.async_copy(src_q.at[colbuf.at[j + 1]],
                                     gbufs.at[1 - slot], sems.at[1 - slot])
                pltpu.async_copy(gbufs.at[slot], acc.at[rowbuf.at[j]],
                                 ssems.at[slot], add=True)
                return 0
            lax.fori_loop(0, kch, chunk_body, 0)
            last = lax.rem(kch - 1, 2)
            pltpu.make_async_copy(gbufs.at[last], acc.at[rowbuf.at[kch - 1]],
                                  ssems.at[last]).wait()

    def elementwise_phase(src_q, dst_state, qid, r, stage1):
        for z in range(NZ):
            la = sid * RPP + z * ZC
            rows = pl.ds(r * PH + la, ZC)
            cpa = pltpu.async_copy(acc.at[pl.ds(la, ZC)], abuf, esems.at[0])
            cps = pltpu.async_copy(src_q.at[rows], sbuf, esems.at[1])
            if not stage1:
                pltpu.async_copy(sig1.at[qid, rows], obuf, osem).wait()
            cpa.wait()
            cps.wait()

            def row_body(i, _):
                d2 = d2v[pl.ds(r * RPP + z * ZC + i, 16)][0]
                for q in range(Q // 16):
                    sl = pl.ds(q * 16, 16)
                    t = abuf[i, sl] + sbuf[i, sl]
                    if stage1:
                        sbuf[i, sl] = C1 * sbuf[i, sl] - (C2 * d2) * t
                    else:
                        sbuf[i, sl] = (CM * d2) * t + ALPHA * obuf[i, sl]
                return 0
            lax.fori_loop(0, ZC, row_body, 0)

            if stage1:
                pltpu.sync_copy(sbuf, sig1.at[qid, rows])
            pltpu.sync_copy(sbuf, dst_state.at[qid, rows])
            pltpu.async_copy(zbuf, acc.at[pl.ds(la, ZC)], zsem)
        for z in range(NZ):
            pltpu.make_async_copy(
                zbuf, acc.at[pl.ds(sid * RPP + z * ZC, ZC)], zsem).wait()

    def sweep(src, dst_state, stage1):
        qid = cid
        for r in range(NPH):
            amul_phase(src.at[qid], r)
            plsc.subcore_barrier()
            elementwise_phase(src.at[qid], dst_state, qid, r, stage1)
            plsc.subcore_barrier()

    sweep(s0h, xstate, stage1=True)

    def iter_body(i, carry):
        @pl.when(lax.rem(i, 2) == 0)
        def _():
            sweep(xstate, xalt, stage1=False)

        @pl.when(lax.rem(i, 2) == 1)
        def _():
            sweep(xalt, xstate, stage1=False)
        return carry
    lax.fori_loop(0, NITER, iter_body, 0)

    pltpu.sync_copy(idxp.at[pl.ds(sid * CHUNK, CHUNK)], idxbuf)
    pltpu.sync_copy(sdegp.at[pl.ds(sid * CHUNK, CHUNK)],
                    sdegv.at[pl.ds(0, CHUNK)])
    pltpu.async_copy(xstate.at[cid].at[idxbuf], gbufs.at[0], osem).wait()

    def out_row(i, _):
        sd = sdegv[pl.ds(i, 16)][0]
        for q in range(Q // 16):
            sl = pl.ds(q * 16, 16)
            gbufs[0, i, sl] = sd * gbufs[0, i, sl]
        return 0
    lax.fori_loop(0, CHUNK, out_row, 0)
    pltpu.sync_copy(gbufs.at[0], outp.at[cid, pl.ds(sid * CHUNK, CHUNK)])


_sc_call = functools.partial(
    pl.kernel,
    out_type=[
        jax.ShapeDtypeStruct((NSC, NP, Q), jnp.float32),
        jax.ShapeDtypeStruct((NSC, NP, Q), jnp.float32),
        jax.ShapeDtypeStruct((NSC, NP, Q), jnp.float32),
        jax.ShapeDtypeStruct((NSC, NIDXP, Q), jnp.float32),
    ],
    mesh=plsc.VectorSubcoreMesh(core_axis_name="c", subcore_axis_name="s",
                                num_cores=NSC),
    scratch_types=[
        pltpu.VMEM((NCB, CHUNK), jnp.int32),
        pltpu.VMEM((NCB, CHUNK), jnp.int32),
        pltpu.VMEM((2, CHUNK, Q), jnp.float32),
        pltpu.VMEM((ZC, Q), jnp.float32),
        pltpu.VMEM((ZC, Q), jnp.float32),
        pltpu.VMEM((ZC, Q), jnp.float32),
        pltpu.VMEM((ZC, Q), jnp.float32),
        pltpu.VMEM((2 * RPP + 16,), jnp.float32),
        pltpu.VMEM((CHUNK + 16,), jnp.float32),
        pltpu.VMEM((4 * NTILE + 16,), jnp.int32),
        pltpu.VMEM((CHUNK,), jnp.int32),
        pltpu.VMEM_SHARED((ACCR, Q), jnp.float32),
        pltpu.SemaphoreType.DMA((2,)),
        pltpu.SemaphoreType.DMA((2,)),
        pltpu.SemaphoreType.DMA((2,)),
        pltpu.SemaphoreType.DMA,
        pltpu.SemaphoreType.DMA,
    ],
)(_sc_body)


def _ceil_div(a, b):
    return (a + b - 1) // b


def kernel(local_preds, idx, edge_index):
    src = edge_index[0].astype(jnp.int32)
    dst = edge_index[1].astype(jnp.int32)

    deg = jax.ops.segment_sum(jnp.ones((E,), jnp.float32), src,
                              num_segments=N) + 1.0
    d2 = 1.0 / deg
    dinv = lax.rsqrt(deg)
    sdeg = deg * dinv

    s0 = dinv[:, None] * local_preds
    s0h = jnp.pad(s0.reshape(N, NSC, Q).transpose(1, 0, 2),
                  ((0, 0), (0, NP - N), (0, 0)))

    phase = (src >= PH).astype(jnp.int32)
    e1 = jnp.sum(phase)
    e0 = E - e1
    c1r = jnp.cumsum(phase) - phase
    c0r = jnp.arange(E, dtype=jnp.int32) - c1r
    rank = jnp.where(phase == 1, c1r, c0r).astype(jnp.int32)
    n0 = SHARE_RND * _ceil_div(e0, NTILE * SHARE_RND)
    n1 = SHARE_RND * _ceil_div(e1, NTILE * SHARE_RND)
    nshare = jnp.where(phase == 1, n1, n0)
    tile = rank // jnp.maximum(nshare, 1)
    within = rank - tile * nshare
    pos = phase * (NTILE * n0) + tile * nshare + within
    colsP = jnp.full((NTOT_CH * CHUNK,), GROW, jnp.int32)
    colsP = colsP.at[pos].add(dst - GROW, unique_indices=True)
    lrowsP = jnp.full((NTOT_CH * CHUNK,), SROW, jnp.int32)
    lrowsP = lrowsP.at[pos].add(src - phase * PH - SROW, unique_indices=True)
    colsa = colsP.reshape(NTOT_CH, CHUNK)
    lrowsa = lrowsP.reshape(NTOT_CH, CHUNK)
    t16 = jnp.arange(NTILE, dtype=jnp.int32)
    c0t = jnp.clip(e0 - t16 * n0, 0, n0)
    c1t = jnp.clip(e1 - t16 * n1, 0, n1)
    counts = _ceil_div(jnp.concatenate([c0t, c1t]), CHUNK).astype(jnp.int32)
    cb0 = t16 * (n0 // CHUNK)
    cb1 = NTILE * (n0 // CHUNK) + t16 * (n1 // CHUNK)
    kcha = jnp.concatenate([counts, cb0, cb1]).astype(jnp.int32)

    d2a = jnp.pad(d2, (0, NP - N))
    idxp = jnp.pad(idx.astype(jnp.int32), (0, NIDXP - N_IDX))
    sdegp = jnp.pad(sdeg[idx], (0, NIDXP - N_IDX))

    _, _, _, outp = _sc_call(s0h, colsa, lrowsa, d2a, kcha, idxp, sdegp)
    return outp.transpose(1, 0, 2).reshape(NIDXP, D)[:N_IDX]
